# baseline (speedup 1.0000x reference)
import functools

import jax
import jax.numpy as jnp
from jax import lax
from jax.experimental import pallas as pl
from jax.experimental.pallas import tpu as pltpu
from jax.experimental.pallas import tpu_sc as plsc

D = 9
R = 4
YR = 256
XR = 256
H = YR * D
W = XR * D
C = 3
ROWS = C * H
NW = 32
RPW = ROWS // NW
CH = 8
NCH = RPW // CH
NCW = W // 16
L = 16


def _floor_f32(t):
    mt = t.astype(jnp.int32)
    mtf = mt.astype(jnp.float32)
    return jnp.where(t < mtf, mt - 1, mt)


def _full(val):
    return jnp.full((L,), val, jnp.int32)


def _sc_body(x_hbm, alpha_hbm, out_hbm,
             al_v, c1_v, c2_v, wx1_v, wx2_v,
             ridx_a, ridx_b, rw_a, rw_b, rows_a, rows_b, out_a, out_b,
             gsem_a, gsem_b, osem_a, osem_b):
    wid = lax.axis_index("c") * 16 + lax.axis_index("s")
    pltpu.sync_copy(alpha_hbm, al_v)
    alpha = al_v[...]
    lane = lax.broadcasted_iota(jnp.int32, (L,), 0)

    def col_body(cw, carry):
        w = cw * L + lane
        xg = lax.div(w, jnp.int32(D))
        u = lax.rem(w, jnp.int32(D))
        t = (R - u).astype(jnp.float32) * alpha
        m = _floor_f32(t)
        f = t - m.astype(jnp.float32)
        sl = pl.ds(cw * L, L)
        c1_v[sl] = jnp.clip(xg + m, 0, XR - 1) * D + u
        c2_v[sl] = jnp.clip(xg + m + 1, 0, XR - 1) * D + u
        wx1_v[sl] = 1.0 - f
        wx2_v[sl] = f
        return carry

    lax.fori_loop(0, NCW, col_body, 0)

    def prep(ch, ridx_v, rw_v, rows_v, gsem):
        @pl.when(ch < NCH)
        def _():
            base = wid * RPW + ch * CH
            i8 = lax.rem(lane, jnp.int32(CH))
            tap = lax.div(lane, jnp.int32(CH))
            g = base + i8
            cc = lax.div(g, jnp.int32(H))
            hh = lax.rem(g, jnp.int32(H))
            y = lax.div(hh, jnp.int32(D))
            v = lax.rem(hh, jnp.int32(D))
            t = (R - v).astype(jnp.float32) * alpha
            m = _floor_f32(t)
            f = t - m.astype(jnp.float32)
            yy = jnp.clip(y + m + tap, 0, YR - 1)
            ridx_v[...] = cc * H + yy * D + v
            rw_v[...] = jnp.where(tap == 0, 1.0 - f, f)
            pltpu.make_async_copy(x_hbm.at[ridx_v], rows_v, gsem).start()

    def work(ch, ridx_v, rw_v, rows_v, outb_v, gsem, osem,
             n_ridx, n_rw, n_rows, n_gsem):
        prep(ch + 1, n_ridx, n_rw, n_rows, n_gsem)
        base = wid * RPW + ch * CH
        pltpu.make_async_copy(x_hbm.at[ridx_v], rows_v, gsem).wait()

        @pl.when(ch >= 2)
        def _():
            prev = base - 2 * CH
            pltpu.make_async_copy(outb_v, out_hbm.at[pl.ds(prev, CH)], osem).wait()

        rw = rw_v[...]
        w1s = [rw.at[_full(i)].get(mode="promise_in_bounds") for i in range(CH)]
        w2s = [rw.at[_full(CH + i)].get(mode="promise_in_bounds") for i in range(CH)]

        def cw_body(cw, icarry):
            sl = pl.ds(cw * L, L)
            c1 = c1_v[sl]
            c2 = c2_v[sl]
            a1 = wx1_v[sl]
            a2 = wx2_v[sl]
            for i in range(CH):
                g11 = plsc.load_gather(rows_v, [_full(i), c1])
                g12 = plsc.load_gather(rows_v, [_full(i), c2])
                g21 = plsc.load_gather(rows_v, [_full(CH + i), c1])
                g22 = plsc.load_gather(rows_v, [_full(CH + i), c2])
                va = w1s[i] * g11 + w2s[i] * g21
                vb = w1s[i] * g12 + w2s[i] * g22
                outb_v[i, sl] = a1 * va + a2 * vb
            return icarry

        lax.fori_loop(0, NCW, cw_body, 0)
        pltpu.make_async_copy(outb_v, out_hbm.at[pl.ds(base, CH)], osem).start()

    prep(jnp.int32(0), ridx_a, rw_a, rows_a, gsem_a)

    def pair_body(i, carry):
        work(2 * i, ridx_a, rw_a, rows_a, out_a, gsem_a, osem_a,
             ridx_b, rw_b, rows_b, gsem_b)

        @pl.when(2 * i + 1 < NCH)
        def _():
            work(2 * i + 1, ridx_b, rw_b, rows_b, out_b, gsem_b, osem_b,
                 ridx_a, rw_a, rows_a, gsem_a)

        return carry

    lax.fori_loop(0, (NCH + 1) // 2, pair_body, 0)

    last_a = wid * RPW + (NCH - 1) * CH
    last_b = wid * RPW + (NCH - 2) * CH
    pltpu.make_async_copy(out_a, out_hbm.at[pl.ds(last_a, CH)], osem_a).wait()
    pltpu.make_async_copy(out_b, out_hbm.at[pl.ds(last_b, CH)], osem_b).wait()


@jax.jit
def kernel(x, alpha):
    x2 = x.reshape(ROWS, W)
    alpha_vec = jnp.full((L,), alpha, jnp.float32)
    run = functools.partial(
        pl.kernel,
        mesh=plsc.VectorSubcoreMesh(core_axis_name="c", subcore_axis_name="s"),
        compiler_params=pltpu.CompilerParams(
            needs_layout_passes=False, use_tc_tiling_on_sc=False),
        out_type=jax.ShapeDtypeStruct((ROWS, W), jnp.float32),
        scratch_types=[
            pltpu.VMEM((L,), jnp.float32),
            pltpu.VMEM((W,), jnp.int32),
            pltpu.VMEM((W,), jnp.int32),
            pltpu.VMEM((W,), jnp.float32),
            pltpu.VMEM((W,), jnp.float32),
            pltpu.VMEM((L,), jnp.int32),
            pltpu.VMEM((L,), jnp.int32),
            pltpu.VMEM((L,), jnp.float32),
            pltpu.VMEM((L,), jnp.float32),
            pltpu.VMEM((2 * CH, W), jnp.float32),
            pltpu.VMEM((2 * CH, W), jnp.float32),
            pltpu.VMEM((CH, W), jnp.float32),
            pltpu.VMEM((CH, W), jnp.float32),
            pltpu.SemaphoreType.DMA,
            pltpu.SemaphoreType.DMA,
            pltpu.SemaphoreType.DMA,
            pltpu.SemaphoreType.DMA,
        ],
    )(_sc_body)
    out = run(x2, alpha_vec)
    return out.reshape(1, C, H, W)


# speedup vs baseline: 1.7950x; 1.7950x over previous
import functools

import jax
import jax.numpy as jnp
from jax import lax
from jax.experimental import pallas as pl
from jax.experimental.pallas import tpu as pltpu
from jax.experimental.pallas import tpu_sc as plsc

D = 9
R = 4
YR = 256
XR = 256
H = YR * D
W = XR * D
C = 3
ROWS = C * H
L = 16

SH = 768
NW = 32
SCROWS = C * SH
RPW = SCROWS // NW
CH = 8
NCH = RPW // CH
NCW = W // L

BR = 48
NB = H // BR
NBT = (H - SH) // BR


def _floor_f32(t):
    mt = t.astype(jnp.int32)
    mtf = mt.astype(jnp.float32)
    return jnp.where(t < mtf, mt - 1, mt)


def _full(val):
    return jnp.full((L,), val, jnp.int32)


def _sc_body(x_hbm, alpha_hbm, out_hbm,
             al_v, c1_v, c2_v, wx1_v, wx2_v,
             ridx_a, ridx_b, rw_a, rw_b, rows_a, rows_b, out_a, out_b,
             gsem_a, gsem_b, osem_a, osem_b):
    wid = lax.axis_index("c") * 16 + lax.axis_index("s")
    pltpu.sync_copy(alpha_hbm, al_v)
    alpha = al_v[...]
    lane = lax.broadcasted_iota(jnp.int32, (L,), 0)

    def col_body(cw, carry):
        w = cw * L + lane
        xg = lax.div(w, jnp.int32(D))
        u = lax.rem(w, jnp.int32(D))
        t = (R - u).astype(jnp.float32) * alpha
        m = _floor_f32(t)
        f = t - m.astype(jnp.float32)
        sl = pl.ds(cw * L, L)
        c1_v[sl] = jnp.clip(xg + m, 0, XR - 1) * D + u
        c2_v[sl] = jnp.clip(xg + m + 1, 0, XR - 1) * D + u
        wx1_v[sl] = 1.0 - f
        wx2_v[sl] = f
        return carry

    lax.fori_loop(0, NCW, col_body, 0)

    def prep(ch, ridx_v, rw_v, rows_v, gsem):
        @pl.when(ch < NCH)
        def _():
            base = wid * RPW + ch * CH
            i8 = lax.rem(lane, jnp.int32(CH))
            tap = lax.div(lane, jnp.int32(CH))
            gl = base + i8
            cc = lax.div(gl, jnp.int32(SH))
            hh = lax.rem(gl, jnp.int32(SH))
            y = lax.div(hh, jnp.int32(D))
            v = lax.rem(hh, jnp.int32(D))
            t = (R - v).astype(jnp.float32) * alpha
            m = _floor_f32(t)
            f = t - m.astype(jnp.float32)
            yy = jnp.clip(y + m + tap, 0, YR - 1)
            ridx_v[...] = cc * H + yy * D + v
            rw_v[...] = jnp.where(tap == 0, 1.0 - f, f)
            pltpu.make_async_copy(x_hbm.at[ridx_v], rows_v, gsem).start()

    def work(ch, ridx_v, rw_v, rows_v, outb_v, gsem, osem,
             n_ridx, n_rw, n_rows, n_gsem):
        prep(ch + 1, n_ridx, n_rw, n_rows, n_gsem)
        base = wid * RPW + ch * CH
        pltpu.make_async_copy(x_hbm.at[ridx_v], rows_v, gsem).wait()

        @pl.when(ch >= 2)
        def _():
            prev = base - 2 * CH
            pltpu.make_async_copy(outb_v, out_hbm.at[pl.ds(prev, CH)], osem).wait()

        rw = rw_v[...]
        w1s = [rw.at[_full(i)].get(mode="promise_in_bounds") for i in range(CH)]
        w2s = [rw.at[_full(CH + i)].get(mode="promise_in_bounds") for i in range(CH)]

        def cw_body(cw, icarry):
            sl = pl.ds(cw * L, L)
            c1 = c1_v[sl]
            c2 = c2_v[sl]
            a1 = wx1_v[sl]
            a2 = wx2_v[sl]
            for i in range(CH):
                g11 = plsc.load_gather(rows_v, [_full(i), c1])
                g12 = plsc.load_gather(rows_v, [_full(i), c2])
                g21 = plsc.load_gather(rows_v, [_full(CH + i), c1])
                g22 = plsc.load_gather(rows_v, [_full(CH + i), c2])
                va = w1s[i] * g11 + w2s[i] * g21
                vb = w1s[i] * g12 + w2s[i] * g22
                outb_v[i, sl] = a1 * va + a2 * vb
            return icarry

        lax.fori_loop(0, NCW, cw_body, 0)
        pltpu.make_async_copy(outb_v, out_hbm.at[pl.ds(base, CH)], osem).start()

    prep(jnp.int32(0), ridx_a, rw_a, rows_a, gsem_a)

    def pair_body(i, carry):
        work(2 * i, ridx_a, rw_a, rows_a, out_a, gsem_a, osem_a,
             ridx_b, rw_b, rows_b, gsem_b)

        @pl.when(2 * i + 1 < NCH)
        def _():
            work(2 * i + 1, ridx_b, rw_b, rows_b, out_b, gsem_b, osem_b,
                 ridx_a, rw_a, rows_a, gsem_a)

        return carry

    lax.fori_loop(0, (NCH + 1) // 2, pair_body, 0)

    last_a = wid * RPW + (NCH - 1) * CH
    last_b = wid * RPW + (NCH - 2) * CH
    pltpu.make_async_copy(out_a, out_hbm.at[pl.ds(last_a, CH)], osem_a).wait()
    pltpu.make_async_copy(out_b, out_hbm.at[pl.ds(last_b, CH)], osem_b).wait()


def _sc_run(x2, alpha_vec):
    run = functools.partial(
        pl.kernel,
        mesh=plsc.VectorSubcoreMesh(core_axis_name="c", subcore_axis_name="s"),
        compiler_params=pltpu.CompilerParams(needs_layout_passes=False),
        out_type=jax.ShapeDtypeStruct((SCROWS, W), jnp.float32),
        scratch_types=[
            pltpu.VMEM((L,), jnp.float32),
            pltpu.VMEM((W,), jnp.int32),
            pltpu.VMEM((W,), jnp.int32),
            pltpu.VMEM((W,), jnp.float32),
            pltpu.VMEM((W,), jnp.float32),
            pltpu.VMEM((L,), jnp.int32),
            pltpu.VMEM((L,), jnp.int32),
            pltpu.VMEM((L,), jnp.float32),
            pltpu.VMEM((L,), jnp.float32),
            pltpu.VMEM((2 * CH, W), jnp.float32),
            pltpu.VMEM((2 * CH, W), jnp.float32),
            pltpu.VMEM((CH, W), jnp.float32),
            pltpu.VMEM((CH, W), jnp.float32),
            pltpu.SemaphoreType.DMA,
            pltpu.SemaphoreType.DMA,
            pltpu.SemaphoreType.DMA,
            pltpu.SemaphoreType.DMA,
        ],
    )(_sc_body)
    return run(x2, alpha_vec)


def _tc_tap_weights(idx_i32, alpha, n_res, off):
    v = idx_i32 % D
    y = idx_i32 // D
    t = -alpha * (v - R).astype(jnp.float32)
    m = jnp.floor(t)
    f = t - m
    mi = m.astype(jnp.int32)
    j1 = jnp.clip(y + mi, 0, n_res - 1) - y
    j2 = jnp.clip(y + mi + 1, 0, n_res - 1) - y
    return jnp.where(j1 == off, 1.0 - f, 0.0) + jnp.where(j2 == off, f, 0.0)


def _tc_body(alpha_ref, prev_ref, cur_ref, next_ref, out_ref, win_ref, pad_ref):
    rb = pl.program_id(1)
    alpha = alpha_ref[0]

    win_ref[0:BR, :] = prev_ref[0, 0]
    win_ref[BR:2 * BR, :] = cur_ref[0, 0]
    win_ref[2 * BR:3 * BR, :] = next_ref[0, 0]

    row = lax.broadcasted_iota(jnp.int32, (BR, 1), 0) + SH + rb * BR
    col = lax.broadcasted_iota(jnp.int32, (1, W), 1)

    tmp = jnp.zeros((BR, W), jnp.float32)
    for off in range(-4, 6):
        b = _tc_tap_weights(row, alpha, YR, off)
        tmp = tmp + b * win_ref[BR + D * off:BR + D * off + BR, :]

    pad_ref[:, 0:BR] = jnp.zeros((BR, BR), jnp.float32)
    pad_ref[:, BR:BR + W] = tmp
    pad_ref[:, BR + W:] = jnp.zeros((BR, BR), jnp.float32)

    out = jnp.zeros((BR, W), jnp.float32)
    for off in range(-4, 6):
        a = _tc_tap_weights(col, alpha, XR, off)
        out = out + a * pad_ref[:, BR + D * off:BR + D * off + W]
    out_ref[0, 0] = out


def _tc_run(x, alpha_arr):
    blk = (1, 1, BR, W)
    rb0 = SH // BR

    def im_prev(c, rb):
        return (0, c, jnp.maximum(rb0 + rb - 1, 0), 0)

    def im_cur(c, rb):
        return (0, c, rb0 + rb, 0)

    def im_next(c, rb):
        return (0, c, jnp.minimum(rb0 + rb + 1, NB - 1), 0)

    def im_out(c, rb):
        return (0, c, rb, 0)

    return pl.pallas_call(
        _tc_body,
        grid=(C, NBT),
        in_specs=[
            pl.BlockSpec(memory_space=pltpu.SMEM),
            pl.BlockSpec(blk, im_prev),
            pl.BlockSpec(blk, im_cur),
            pl.BlockSpec(blk, im_next),
        ],
        out_specs=pl.BlockSpec(blk, im_out),
        out_shape=jax.ShapeDtypeStruct((1, C, H - SH, W), jnp.float32),
        scratch_shapes=[
            pltpu.VMEM((3 * BR, W), jnp.float32),
            pltpu.VMEM((BR, W + 2 * BR), jnp.float32),
        ],
    )(alpha_arr, x, x, x)


@jax.jit
def kernel(x, alpha):
    alpha32 = alpha.astype(jnp.float32)
    sc_out = _sc_run(x.reshape(ROWS, W), jnp.full((L,), alpha32, jnp.float32))
    tc_out = _tc_run(x, jnp.reshape(alpha32, (1,)))
    top = sc_out.reshape(C, SH, W)
    bottom = tc_out.reshape(C, H - SH, W)
    return jnp.concatenate([top, bottom], axis=1).reshape(1, C, H, W)


# speedup vs baseline: 2.4014x; 1.3379x over previous
import functools

import jax
import jax.numpy as jnp
from jax import lax
from jax.experimental import pallas as pl
from jax.experimental.pallas import tpu as pltpu
from jax.experimental.pallas import tpu_sc as plsc

D = 9
R = 4
YR = 256
XR = 256
H = YR * D
W = XR * D
C = 3
ROWS = C * H
L = 16

SH = 768
NW = 32
SCROWS = C * SH
RPW = SCROWS // NW
CH = 8
NCH = RPW // CH
NCW = W // L

BR = 48
NB = H // BR
NBT = (H - SH) // BR


def _floor_f32(t):
    mt = t.astype(jnp.int32)
    mtf = mt.astype(jnp.float32)
    return jnp.where(t < mtf, mt - 1, mt)


def _full(val):
    return jnp.full((L,), val, jnp.int32)


def _sc_body(x_hbm, alpha_hbm, out_hbm,
             al_v, c1_v, c2_v, wx1_v, wx2_v,
             ridx_a, ridx_b, rw_a, rw_b, rows_a, rows_b, out_a, out_b,
             gsem_a, gsem_b, osem_a, osem_b):
    wid = lax.axis_index("c") * 16 + lax.axis_index("s")
    pltpu.sync_copy(alpha_hbm, al_v)
    alpha = al_v[...]
    lane = lax.broadcasted_iota(jnp.int32, (L,), 0)

    def col_body(cw, carry):
        w = cw * L + lane
        xg = lax.div(w, jnp.int32(D))
        u = lax.rem(w, jnp.int32(D))
        t = (R - u).astype(jnp.float32) * alpha
        m = _floor_f32(t)
        f = t - m.astype(jnp.float32)
        sl = pl.ds(cw * L, L)
        c1_v[sl] = jnp.clip(xg + m, 0, XR - 1) * D + u
        c2_v[sl] = jnp.clip(xg + m + 1, 0, XR - 1) * D + u
        wx1_v[sl] = 1.0 - f
        wx2_v[sl] = f
        return carry

    lax.fori_loop(0, NCW, col_body, 0)

    def prep(ch, ridx_v, rw_v, rows_v, gsem):
        @pl.when(ch < NCH)
        def _():
            base = wid * RPW + ch * CH
            i8 = lax.rem(lane, jnp.int32(CH))
            tap = lax.div(lane, jnp.int32(CH))
            gl = base + i8
            cc = lax.div(gl, jnp.int32(SH))
            hh = lax.rem(gl, jnp.int32(SH))
            y = lax.div(hh, jnp.int32(D))
            v = lax.rem(hh, jnp.int32(D))
            t = (R - v).astype(jnp.float32) * alpha
            m = _floor_f32(t)
            f = t - m.astype(jnp.float32)
            yy = jnp.clip(y + m + tap, 0, YR - 1)
            ridx_v[...] = cc * H + yy * D + v
            rw_v[...] = jnp.where(tap == 0, 1.0 - f, f)
            pltpu.make_async_copy(x_hbm.at[ridx_v], rows_v, gsem).start()

    def work(ch, ridx_v, rw_v, rows_v, outb_v, gsem, osem,
             n_ridx, n_rw, n_rows, n_gsem):
        prep(ch + 1, n_ridx, n_rw, n_rows, n_gsem)
        base = wid * RPW + ch * CH
        pltpu.make_async_copy(x_hbm.at[ridx_v], rows_v, gsem).wait()

        @pl.when(ch >= 2)
        def _():
            prev = base - 2 * CH
            pltpu.make_async_copy(outb_v, out_hbm.at[pl.ds(prev, CH)], osem).wait()

        rw = rw_v[...]
        w1s = [rw.at[_full(i)].get(mode="promise_in_bounds") for i in range(CH)]
        w2s = [rw.at[_full(CH + i)].get(mode="promise_in_bounds") for i in range(CH)]

        def cw_body(cw, icarry):
            sl = pl.ds(cw * L, L)
            c1 = c1_v[sl]
            c2 = c2_v[sl]
            a1 = wx1_v[sl]
            a2 = wx2_v[sl]
            for i in range(CH):
                g11 = plsc.load_gather(rows_v, [_full(i), c1])
                g12 = plsc.load_gather(rows_v, [_full(i), c2])
                g21 = plsc.load_gather(rows_v, [_full(CH + i), c1])
                g22 = plsc.load_gather(rows_v, [_full(CH + i), c2])
                va = w1s[i] * g11 + w2s[i] * g21
                vb = w1s[i] * g12 + w2s[i] * g22
                outb_v[i, sl] = a1 * va + a2 * vb
            return icarry

        lax.fori_loop(0, NCW, cw_body, 0)
        pltpu.make_async_copy(outb_v, out_hbm.at[pl.ds(base, CH)], osem).start()

    prep(jnp.int32(0), ridx_a, rw_a, rows_a, gsem_a)

    def pair_body(i, carry):
        work(2 * i, ridx_a, rw_a, rows_a, out_a, gsem_a, osem_a,
             ridx_b, rw_b, rows_b, gsem_b)

        @pl.when(2 * i + 1 < NCH)
        def _():
            work(2 * i + 1, ridx_b, rw_b, rows_b, out_b, gsem_b, osem_b,
                 ridx_a, rw_a, rows_a, gsem_a)

        return carry

    lax.fori_loop(0, (NCH + 1) // 2, pair_body, 0)

    last_a = wid * RPW + (NCH - 1) * CH
    last_b = wid * RPW + (NCH - 2) * CH
    pltpu.make_async_copy(out_a, out_hbm.at[pl.ds(last_a, CH)], osem_a).wait()
    pltpu.make_async_copy(out_b, out_hbm.at[pl.ds(last_b, CH)], osem_b).wait()


def _sc_run(x2, alpha_vec):
    run = functools.partial(
        pl.kernel,
        mesh=plsc.VectorSubcoreMesh(core_axis_name="c", subcore_axis_name="s"),
        compiler_params=pltpu.CompilerParams(needs_layout_passes=False),
        out_type=jax.ShapeDtypeStruct((SCROWS, W), jnp.float32),
        scratch_types=[
            pltpu.VMEM((L,), jnp.float32),
            pltpu.VMEM((W,), jnp.int32),
            pltpu.VMEM((W,), jnp.int32),
            pltpu.VMEM((W,), jnp.float32),
            pltpu.VMEM((W,), jnp.float32),
            pltpu.VMEM((L,), jnp.int32),
            pltpu.VMEM((L,), jnp.int32),
            pltpu.VMEM((L,), jnp.float32),
            pltpu.VMEM((L,), jnp.float32),
            pltpu.VMEM((2 * CH, W), jnp.float32),
            pltpu.VMEM((2 * CH, W), jnp.float32),
            pltpu.VMEM((CH, W), jnp.float32),
            pltpu.VMEM((CH, W), jnp.float32),
            pltpu.SemaphoreType.DMA,
            pltpu.SemaphoreType.DMA,
            pltpu.SemaphoreType.DMA,
            pltpu.SemaphoreType.DMA,
        ],
    )(_sc_body)
    return run(x2, alpha_vec)


def _tc_tap_weights(idx_i32, alpha, n_res, off):
    v = idx_i32 % D
    y = idx_i32 // D
    t = -alpha * (v - R).astype(jnp.float32)
    m = jnp.floor(t)
    f = t - m
    mi = m.astype(jnp.int32)
    j1 = jnp.clip(y + mi, 0, n_res - 1) - y
    j2 = jnp.clip(y + mi + 1, 0, n_res - 1) - y
    return jnp.where(j1 == off, 1.0 - f, 0.0) + jnp.where(j2 == off, f, 0.0)


def _tc_body(alpha_ref, prev_ref, cur_ref, next_ref, out_ref, win_ref, pad_ref):
    rb = pl.program_id(1)
    alpha = alpha_ref[0]

    win_ref[0:BR, :] = prev_ref[0, 0]
    win_ref[BR:2 * BR, :] = cur_ref[0, 0]
    win_ref[2 * BR:3 * BR, :] = next_ref[0, 0]

    row = lax.broadcasted_iota(jnp.int32, (BR, 1), 0) + SH + rb * BR
    col = lax.broadcasted_iota(jnp.int32, (1, W), 1)

    tmp = jnp.zeros((BR, W), jnp.float32)
    for off in range(-2, 4):
        b = _tc_tap_weights(row, alpha, YR, off)
        tmp = tmp + b * win_ref[BR + D * off:BR + D * off + BR, :]

    pad_ref[:, 0:BR] = jnp.zeros((BR, BR), jnp.float32)
    pad_ref[:, BR:BR + W] = tmp
    pad_ref[:, BR + W:] = jnp.zeros((BR, BR), jnp.float32)

    out = jnp.zeros((BR, W), jnp.float32)
    for off in range(-2, 4):
        a = _tc_tap_weights(col, alpha, XR, off)
        out = out + a * pad_ref[:, BR + D * off:BR + D * off + W]
    out_ref[0, 0] = out


def _tc_run(x, alpha_arr):
    blk = (1, 1, BR, W)
    rb0 = SH // BR

    def im_prev(c, rb):
        return (0, c, jnp.maximum(rb0 + rb - 1, 0), 0)

    def im_cur(c, rb):
        return (0, c, rb0 + rb, 0)

    def im_next(c, rb):
        return (0, c, jnp.minimum(rb0 + rb + 1, NB - 1), 0)

    def im_out(c, rb):
        return (0, c, rb, 0)

    return pl.pallas_call(
        _tc_body,
        grid=(C, NBT),
        in_specs=[
            pl.BlockSpec(memory_space=pltpu.SMEM),
            pl.BlockSpec(blk, im_prev),
            pl.BlockSpec(blk, im_cur),
            pl.BlockSpec(blk, im_next),
        ],
        out_specs=pl.BlockSpec(blk, im_out),
        out_shape=jax.ShapeDtypeStruct((1, C, H - SH, W), jnp.float32),
        scratch_shapes=[
            pltpu.VMEM((3 * BR, W), jnp.float32),
            pltpu.VMEM((BR, W + 2 * BR), jnp.float32),
        ],
    )(alpha_arr, x, x, x)


@jax.jit
def kernel(x, alpha):
    alpha32 = alpha.astype(jnp.float32)
    sc_out = _sc_run(x.reshape(ROWS, W), jnp.full((L,), alpha32, jnp.float32))
    tc_out = _tc_run(x, jnp.reshape(alpha32, (1,)))
    top = sc_out.reshape(C, SH, W)
    bottom = tc_out.reshape(C, H - SH, W)
    return jnp.concatenate([top, bottom], axis=1).reshape(1, C, H, W)


# speedup vs baseline: 2.7151x; 1.1306x over previous
import functools

import jax
import jax.numpy as jnp
from jax import lax
from jax.experimental import pallas as pl
from jax.experimental.pallas import tpu as pltpu
from jax.experimental.pallas import tpu_sc as plsc

D = 9
R = 4
YR = 256
XR = 256
H = YR * D
W = XR * D
C = 3
ROWS = C * H
L = 16

SH = 768
NW = 32
SCROWS = C * SH
RPW = SCROWS // NW
CH = 8
NCH = RPW // CH
NCW = W // L

BR = 48
NB = H // BR
NBT = (H - SH) // BR


def _floor_f32(t):
    mt = t.astype(jnp.int32)
    mtf = mt.astype(jnp.float32)
    return jnp.where(t < mtf, mt - 1, mt)


def _full(val):
    return jnp.full((L,), val, jnp.int32)


def _sc_body(x_hbm, alpha_hbm, out_hbm,
             al_v, c1_v, c2_v, wx1_v, wx2_v,
             ridx_a, ridx_b, rw_a, rw_b, rows_a, rows_b, out_a, out_b,
             gsem_a, gsem_b, osem_a, osem_b):
    wid = lax.axis_index("c") * 16 + lax.axis_index("s")
    pltpu.sync_copy(alpha_hbm, al_v)
    alpha = al_v[...]
    lane = lax.broadcasted_iota(jnp.int32, (L,), 0)

    def col_body(cw, carry):
        w = cw * L + lane
        xg = lax.div(w, jnp.int32(D))
        u = lax.rem(w, jnp.int32(D))
        t = (R - u).astype(jnp.float32) * alpha
        m = _floor_f32(t)
        f = t - m.astype(jnp.float32)
        sl = pl.ds(cw * L, L)
        c1_v[sl] = jnp.clip(xg + m, 0, XR - 1) * D + u
        c2_v[sl] = jnp.clip(xg + m + 1, 0, XR - 1) * D + u
        wx1_v[sl] = 1.0 - f
        wx2_v[sl] = f
        return carry

    lax.fori_loop(0, NCW, col_body, 0)

    def prep(ch, ridx_v, rw_v, rows_v, gsem):
        @pl.when(ch < NCH)
        def _():
            base = wid * RPW + ch * CH
            i8 = lax.rem(lane, jnp.int32(CH))
            tap = lax.div(lane, jnp.int32(CH))
            gl = base + i8
            cc = lax.div(gl, jnp.int32(SH))
            hh = lax.rem(gl, jnp.int32(SH))
            y = lax.div(hh, jnp.int32(D))
            v = lax.rem(hh, jnp.int32(D))
            t = (R - v).astype(jnp.float32) * alpha
            m = _floor_f32(t)
            f = t - m.astype(jnp.float32)
            yy = jnp.clip(y + m + tap, 0, YR - 1)
            ridx_v[...] = cc * H + yy * D + v
            rw_v[...] = jnp.where(tap == 0, 1.0 - f, f)
            pltpu.make_async_copy(x_hbm.at[ridx_v], rows_v, gsem).start()

    def work(ch, ridx_v, rw_v, rows_v, outb_v, gsem, osem,
             n_ridx, n_rw, n_rows, n_gsem):
        prep(ch + 1, n_ridx, n_rw, n_rows, n_gsem)
        base = wid * RPW + ch * CH
        pltpu.make_async_copy(x_hbm.at[ridx_v], rows_v, gsem).wait()

        @pl.when(ch >= 2)
        def _():
            prev = base - 2 * CH
            pltpu.make_async_copy(outb_v, out_hbm.at[pl.ds(prev, CH)], osem).wait()

        rw = rw_v[...]
        w1s = [rw.at[_full(i)].get(mode="promise_in_bounds") for i in range(CH)]
        w2s = [rw.at[_full(CH + i)].get(mode="promise_in_bounds") for i in range(CH)]

        def cw_body(cw, icarry):
            sl = pl.ds(cw * L, L)
            c1 = c1_v[sl]
            c2 = c2_v[sl]
            a1 = wx1_v[sl]
            a2 = wx2_v[sl]
            for i in range(CH):
                g11 = plsc.load_gather(rows_v, [_full(i), c1])
                g12 = plsc.load_gather(rows_v, [_full(i), c2])
                g21 = plsc.load_gather(rows_v, [_full(CH + i), c1])
                g22 = plsc.load_gather(rows_v, [_full(CH + i), c2])
                va = w1s[i] * g11 + w2s[i] * g21
                vb = w1s[i] * g12 + w2s[i] * g22
                outb_v[i, sl] = a1 * va + a2 * vb
            return icarry

        lax.fori_loop(0, NCW, cw_body, 0)
        pltpu.make_async_copy(outb_v, out_hbm.at[pl.ds(base, CH)], osem).start()

    prep(jnp.int32(0), ridx_a, rw_a, rows_a, gsem_a)

    def pair_body(i, carry):
        work(2 * i, ridx_a, rw_a, rows_a, out_a, gsem_a, osem_a,
             ridx_b, rw_b, rows_b, gsem_b)

        @pl.when(2 * i + 1 < NCH)
        def _():
            work(2 * i + 1, ridx_b, rw_b, rows_b, out_b, gsem_b, osem_b,
                 ridx_a, rw_a, rows_a, gsem_a)

        return carry

    lax.fori_loop(0, (NCH + 1) // 2, pair_body, 0)

    last_a = wid * RPW + (NCH - 1) * CH
    last_b = wid * RPW + (NCH - 2) * CH
    pltpu.make_async_copy(out_a, out_hbm.at[pl.ds(last_a, CH)], osem_a).wait()
    pltpu.make_async_copy(out_b, out_hbm.at[pl.ds(last_b, CH)], osem_b).wait()


def _sc_run(x2, alpha_vec):
    run = functools.partial(
        pl.kernel,
        mesh=plsc.VectorSubcoreMesh(core_axis_name="c", subcore_axis_name="s"),
        compiler_params=pltpu.CompilerParams(needs_layout_passes=False),
        out_type=jax.ShapeDtypeStruct((SCROWS, W), jnp.float32),
        scratch_types=[
            pltpu.VMEM((L,), jnp.float32),
            pltpu.VMEM((W,), jnp.int32),
            pltpu.VMEM((W,), jnp.int32),
            pltpu.VMEM((W,), jnp.float32),
            pltpu.VMEM((W,), jnp.float32),
            pltpu.VMEM((L,), jnp.int32),
            pltpu.VMEM((L,), jnp.int32),
            pltpu.VMEM((L,), jnp.float32),
            pltpu.VMEM((L,), jnp.float32),
            pltpu.VMEM((2 * CH, W), jnp.float32),
            pltpu.VMEM((2 * CH, W), jnp.float32),
            pltpu.VMEM((CH, W), jnp.float32),
            pltpu.VMEM((CH, W), jnp.float32),
            pltpu.SemaphoreType.DMA,
            pltpu.SemaphoreType.DMA,
            pltpu.SemaphoreType.DMA,
            pltpu.SemaphoreType.DMA,
        ],
    )(_sc_body)
    return run(x2, alpha_vec)


def _tc_tap_weights(idx_i32, alpha, n_res, off):
    v = idx_i32 % D
    y = idx_i32 // D
    t = -alpha * (v - R).astype(jnp.float32)
    m = jnp.floor(t)
    f = t - m
    mi = m.astype(jnp.int32)
    j1 = jnp.clip(y + mi, 0, n_res - 1) - y
    j2 = jnp.clip(y + mi + 1, 0, n_res - 1) - y
    return jnp.where(j1 == off, 1.0 - f, 0.0) + jnp.where(j2 == off, f, 0.0)


def _tc_body(alpha_ref, prev_ref, cur_ref, next_ref, out_ref, win_ref, pad_ref):
    rb = pl.program_id(1)
    alpha = alpha_ref[0]

    win_ref[0:BR, :] = prev_ref[0, 0]
    win_ref[BR:2 * BR, :] = cur_ref[0, 0]
    win_ref[2 * BR:3 * BR, :] = next_ref[0, 0]

    row = lax.broadcasted_iota(jnp.int32, (BR, 1), 0) + SH + rb * BR
    col = lax.broadcasted_iota(jnp.int32, (1, W), 1)

    tmp = jnp.zeros((BR, W), jnp.float32)
    for off in range(-2, 4):
        b = _tc_tap_weights(row, alpha, YR, off)
        tmp = tmp + b * win_ref[BR + D * off:BR + D * off + BR, :]

    pad_ref[:, 0:BR] = jnp.zeros((BR, BR), jnp.float32)
    pad_ref[:, BR:BR + W] = tmp
    pad_ref[:, BR + W:] = jnp.zeros((BR, BR), jnp.float32)

    out = jnp.zeros((BR, W), jnp.float32)
    for off in range(-2, 4):
        a = _tc_tap_weights(col, alpha, XR, off)
        out = out + a * pad_ref[:, BR + D * off:BR + D * off + W]
    out_ref[0, 0] = out


def _tc_run(x, alpha_arr):
    blk = (1, 1, BR, W)
    rb0 = SH // BR

    def im_prev(c, rb):
        return (0, c, jnp.maximum(rb0 + rb - 1, 0), 0)

    def im_cur(c, rb):
        return (0, c, rb0 + rb, 0)

    def im_next(c, rb):
        return (0, c, jnp.minimum(rb0 + rb + 1, NB - 1), 0)

    return pl.pallas_call(
        _tc_body,
        grid=(C, NBT),
        in_specs=[
            pl.BlockSpec(memory_space=pltpu.SMEM),
            pl.BlockSpec(blk, im_prev),
            pl.BlockSpec(blk, im_cur),
            pl.BlockSpec(blk, im_next),
        ],
        out_specs=pl.BlockSpec(blk, im_cur),
        out_shape=jax.ShapeDtypeStruct((1, C, H, W), jnp.float32),
        scratch_shapes=[
            pltpu.VMEM((3 * BR, W), jnp.float32),
            pltpu.VMEM((BR, W + 2 * BR), jnp.float32),
        ],
    )(alpha_arr, x, x, x)


@jax.jit
def kernel(x, alpha):
    alpha32 = alpha.astype(jnp.float32)
    sc_out = _sc_run(x.reshape(ROWS, W), jnp.full((L,), alpha32, jnp.float32))
    tc_out = _tc_run(x, jnp.reshape(alpha32, (1,)))
    return lax.dynamic_update_slice(
        tc_out, sc_out.reshape(1, C, SH, W), (0, 0, 0, 0))
